# baseline (speedup 1.0000x reference)
import functools

import jax
import jax.numpy as jnp
from jax import lax
from jax.experimental import pallas as pl
from jax.experimental.pallas import tpu as pltpu
from jax.experimental.pallas import tpu_sc as plsc

NC = 2
NS = 16
NW = NC * NS
CHUNK = 64
NBUF = 8


@jax.jit
def kernel(input_ids, weight):
    B, S = input_ids.shape
    V, D = weight.shape
    total = B * S
    rows_per_w = total // NW
    n_chunks = rows_per_w // CHUNK
    assert rows_per_w * NW == total and n_chunks * CHUNK == rows_per_w
    assert (n_chunks - NBUF) % NBUF == 0

    mesh = plsc.VectorSubcoreMesh(core_axis_name="c", subcore_axis_name="s")

    idx3 = input_ids.reshape(NW, n_chunks, CHUNK).astype(jnp.int32)

    @functools.partial(
        pl.kernel,
        out_type=jax.ShapeDtypeStruct((total, D), jnp.float32),
        mesh=mesh,
        scratch_types=[
            pltpu.VMEM((n_chunks, CHUNK), jnp.int32),
            pltpu.VMEM((NBUF, CHUNK, D), jnp.float32),
            pltpu.SemaphoreType.DMA((NBUF,)),
            pltpu.SemaphoreType.DMA((NBUF,)),
        ],
    )
    def run(idx_hbm, w_hbm, out_hbm, idx_v, rows_v, gsem, osem):
        wid = lax.axis_index("s") * NC + lax.axis_index("c")
        base = wid * rows_per_w
        pltpu.sync_copy(idx_hbm.at[wid], idx_v)

        def start_gather(b, j):
            pltpu.async_copy(w_hbm.at[idx_v.at[j]], rows_v.at[b], gsem.at[b])

        def wait_gather(b, j):
            pltpu.make_async_copy(
                w_hbm.at[idx_v.at[j]], rows_v.at[b], gsem.at[b]
            ).wait()

        def start_write(b, j):
            pltpu.async_copy(
                rows_v.at[b], out_hbm.at[pl.ds(base + j * CHUNK, CHUNK)],
                osem.at[b])

        def wait_write(b, j):
            pltpu.make_async_copy(
                rows_v.at[b], out_hbm.at[pl.ds(base + j * CHUNK, CHUNK)],
                osem.at[b]).wait()

        for b in range(NBUF):
            start_gather(b, b)

        @pl.loop(0, n_chunks - NBUF, step=NBUF)
        def blk(t):
            for b in range(NBUF):
                wait_gather(b, t + b)
                start_write(b, t + b)
            for b in range(NBUF):
                wait_write(b, t + b)
                start_gather(b, t + b + NBUF)

        t0 = n_chunks - NBUF
        for b in range(NBUF):
            wait_gather(b, t0 + b)
            start_write(b, t0 + b)
        for b in range(NBUF):
            wait_write(b, t0 + b)

    out = run(idx3, weight)
    return out.reshape(B, S, D)


# speedup vs baseline: 1.0046x; 1.0046x over previous
import functools

import jax
import jax.numpy as jnp
from jax import lax
from jax.experimental import pallas as pl
from jax.experimental.pallas import tpu as pltpu
from jax.experimental.pallas import tpu_sc as plsc

NC = 2
NS = 16
NW = NC * NS
CHUNK = 128
NBUF = 4


@jax.jit
def kernel(input_ids, weight):
    B, S = input_ids.shape
    V, D = weight.shape
    total = B * S
    rows_per_w = total // NW
    n_chunks = rows_per_w // CHUNK
    assert rows_per_w * NW == total and n_chunks * CHUNK == rows_per_w
    assert (n_chunks - NBUF) % NBUF == 0

    mesh = plsc.VectorSubcoreMesh(core_axis_name="c", subcore_axis_name="s")

    idx3 = input_ids.reshape(NW, n_chunks, CHUNK).astype(jnp.int32)

    @functools.partial(
        pl.kernel,
        out_type=jax.ShapeDtypeStruct((total, D), jnp.float32),
        mesh=mesh,
        scratch_types=[
            pltpu.VMEM((n_chunks, CHUNK), jnp.int32),
            pltpu.VMEM((NBUF, CHUNK, D), jnp.float32),
            pltpu.SemaphoreType.DMA((NBUF,)),
            pltpu.SemaphoreType.DMA((NBUF,)),
        ],
    )
    def run(idx_hbm, w_hbm, out_hbm, idx_v, rows_v, gsem, osem):
        wid = lax.axis_index("s") * NC + lax.axis_index("c")
        base = wid * rows_per_w
        pltpu.sync_copy(idx_hbm.at[wid], idx_v)

        def start_gather(b, j):
            pltpu.async_copy(w_hbm.at[idx_v.at[j]], rows_v.at[b], gsem.at[b])

        def wait_gather(b, j):
            pltpu.make_async_copy(
                w_hbm.at[idx_v.at[j]], rows_v.at[b], gsem.at[b]
            ).wait()

        def start_write(b, j):
            pltpu.async_copy(
                rows_v.at[b], out_hbm.at[pl.ds(base + j * CHUNK, CHUNK)],
                osem.at[b])

        def wait_write(b, j):
            pltpu.make_async_copy(
                rows_v.at[b], out_hbm.at[pl.ds(base + j * CHUNK, CHUNK)],
                osem.at[b]).wait()

        for b in range(NBUF):
            start_gather(b, b)

        @pl.loop(0, n_chunks - NBUF, step=NBUF)
        def blk(t):
            for b in range(NBUF):
                wait_gather(b, t + b)
                start_write(b, t + b)
            for b in range(NBUF):
                wait_write(b, t + b)
                start_gather(b, t + b + NBUF)

        t0 = n_chunks - NBUF
        for b in range(NBUF):
            wait_gather(b, t0 + b)
            start_write(b, t0 + b)
        for b in range(NBUF):
            wait_write(b, t0 + b)

    out = run(idx3, weight)
    return out.reshape(B, S, D)


# speedup vs baseline: 1.0112x; 1.0066x over previous
import functools

import jax
import jax.numpy as jnp
from jax import lax
from jax.experimental import pallas as pl
from jax.experimental.pallas import tpu as pltpu
from jax.experimental.pallas import tpu_sc as plsc

NC = 2
NS = 16
NW = NC * NS
CHUNK = 64
G = 4
P = 2 * G


@jax.jit
def kernel(input_ids, weight):
    B, S = input_ids.shape
    V, D = weight.shape
    total = B * S
    rows_per_w = total // NW
    n_chunks = rows_per_w // CHUNK
    assert rows_per_w * NW == total and n_chunks * CHUNK == rows_per_w
    assert n_chunks % P == 0 and n_chunks >= 2 * P

    mesh = plsc.VectorSubcoreMesh(core_axis_name="c", subcore_axis_name="s")

    idx3 = input_ids.reshape(NW, n_chunks, CHUNK).astype(jnp.int32)

    @functools.partial(
        pl.kernel,
        out_type=jax.ShapeDtypeStruct((total, D), jnp.float32),
        mesh=mesh,
        scratch_types=[
            pltpu.VMEM((n_chunks, CHUNK), jnp.int32),
            pltpu.VMEM((P, CHUNK, D), jnp.float32),
            pltpu.SemaphoreType.DMA((P,)),
            pltpu.SemaphoreType.DMA((P,)),
        ],
    )
    def run(idx_hbm, w_hbm, out_hbm, idx_v, rows_v, gsem, osem):
        wid = lax.axis_index("s") * NC + lax.axis_index("c")
        base = wid * rows_per_w
        pltpu.sync_copy(idx_hbm.at[wid], idx_v)

        def start_gather(b, j):
            pltpu.async_copy(w_hbm.at[idx_v.at[j]], rows_v.at[b], gsem.at[b])

        def wait_gather(b, j):
            pltpu.make_async_copy(
                w_hbm.at[idx_v.at[j]], rows_v.at[b], gsem.at[b]
            ).wait()

        def start_write(b, j):
            pltpu.async_copy(
                rows_v.at[b], out_hbm.at[pl.ds(base + j * CHUNK, CHUNK)],
                osem.at[b])

        def wait_write(b, j):
            pltpu.make_async_copy(
                rows_v.at[b], out_hbm.at[pl.ds(base + j * CHUNK, CHUNK)],
                osem.at[b]).wait()

        def block(t, first=False, last=False):
            for u in range(P):
                wait_gather(u, t + u)
                start_write(u, t + u)
                if first and u < G:
                    continue
                v = (u - G) % P
                jw = t + u - G
                wait_write(v, jw)
                if not last:
                    start_gather(v, jw + P)
                elif u < G:
                    start_gather(v, jw + P)

        for b in range(P):
            start_gather(b, b)

        block(0, first=True)

        @pl.loop(P, n_chunks - P, step=P)
        def steady(t):
            block(t)

        t_last = n_chunks - P
        block(t_last, last=True)
        for b in range(G, P):
            wait_write(b, t_last + b)

    out = run(idx3, weight)
    return out.reshape(B, S, D)
